# Initial kernel scaffold; baseline (speedup 1.0000x reference)
import jax
import jax.numpy as jnp
from jax.experimental import pallas as pl


def kernel(src, trg, Wq, Wk, Wv):
    raise NotImplementedError("write your pallas kernel here")


# speedup vs baseline: 4.6604x; 4.6604x over previous
import jax
import jax.numpy as jnp
from jax.experimental import pallas as pl
from jax.experimental.pallas import tpu as pltpu

_EPS = 1e-5
_QBLK = 512


def _can_kernel(src_ref, trg_ref, wq_ref, wk_ref, wv_ref, out_ref):
    src = src_ref[0]
    trg = trg_ref[0]
    C, Ts = src.shape
    Tt = trg.shape[1]

    def inorm(x, n):
        mu = jnp.mean(x, axis=1, keepdims=True)
        d = x - mu
        var = jnp.sum(d * d, axis=1, keepdims=True) * (1.0 / (n - 1))
        return d / (jnp.sqrt(var) + _EPS)

    src_n = inorm(src, Ts)
    trg_n = inorm(trg, Tt)

    dn_proj = (((0,), (1,)), ((), ()))
    q = jax.lax.dot_general(src_n, wq_ref[...], dn_proj,
                            preferred_element_type=jnp.float32)
    k = jax.lax.dot_general(trg_n, wk_ref[...], dn_proj,
                            preferred_element_type=jnp.float32)
    v = jax.lax.dot_general(trg, wv_ref[...], dn_proj,
                            preferred_element_type=jnp.float32)
    v2 = v * v
    q = q * (1.0 / jnp.sqrt(jnp.float32(C)))

    dn_qkt = (((1,), (1,)), ((), ()))
    acc_mu = jnp.zeros((1, C), jnp.float32)
    acc_var = jnp.zeros((1, C), jnp.float32)
    for i in range(Ts // _QBLK):
        qb = q[i * _QBLK:(i + 1) * _QBLK]
        s = jax.lax.dot_general(qb, k, dn_qkt,
                                preferred_element_type=jnp.float32)
        s = s - jnp.max(s, axis=1, keepdims=True)
        e = jnp.exp(s)
        p = e * (1.0 / jnp.sum(e, axis=1, keepdims=True))
        m = jnp.dot(p, v, preferred_element_type=jnp.float32)
        e2 = jnp.dot(p, v2, preferred_element_type=jnp.float32)
        acc_mu = acc_mu + jnp.sum(m, axis=0, keepdims=True)
        acc_var = acc_var + jnp.sum(jnp.maximum(e2 - m * m, 0.0), axis=0,
                                    keepdims=True)

    mu = acc_mu * (1.0 / Ts)
    std = jnp.sqrt(acc_var * (1.0 / Ts))
    out_ref[0] = std.reshape(C, 1) * src_n + mu.reshape(C, 1)


def kernel(src, trg, Wq, Wk, Wv):
    B, C, Ts = src.shape
    Tt = trg.shape[2]
    return pl.pallas_call(
        _can_kernel,
        out_shape=jax.ShapeDtypeStruct((B, C, Ts), src.dtype),
        grid=(B,),
        in_specs=[
            pl.BlockSpec((1, C, Ts), lambda b: (b, 0, 0)),
            pl.BlockSpec((1, C, Tt), lambda b: (b, 0, 0)),
            pl.BlockSpec((C, C), lambda b: (0, 0)),
            pl.BlockSpec((C, C), lambda b: (0, 0)),
            pl.BlockSpec((C, C), lambda b: (0, 0)),
        ],
        out_specs=pl.BlockSpec((1, C, Ts), lambda b: (b, 0, 0)),
        compiler_params=pltpu.CompilerParams(
            dimension_semantics=("parallel",),
            vmem_limit_bytes=64 * 1024 * 1024,
        ),
        name="chan_adaptive_norm",
    )(src, trg, Wq, Wk, Wv)


# speedup vs baseline: 5.1753x; 1.1105x over previous
import jax
import jax.numpy as jnp
from jax.experimental import pallas as pl
from jax.experimental.pallas import tpu as pltpu

_EPS = 1e-5
_QBLK = 512


def _can_kernel(src_ref, trg_ref, wq_ref, wk_ref, wv_ref, out_ref):
    src = src_ref[0]
    trg = trg_ref[0]
    C, Ts = src.shape
    Tt = trg.shape[1]

    def inorm(x, n):
        mu = jnp.mean(x, axis=1, keepdims=True)
        d = x - mu
        var = jnp.sum(d * d, axis=1, keepdims=True) * (1.0 / (n - 1))
        return d / (jnp.sqrt(var) + _EPS)

    src_n = inorm(src, Ts)
    trg_n = inorm(trg, Tt)

    dn_proj = (((0,), (1,)), ((), ()))
    q = jax.lax.dot_general(src_n, wq_ref[...], dn_proj,
                            preferred_element_type=jnp.float32)
    k = jax.lax.dot_general(trg_n, wk_ref[...], dn_proj,
                            preferred_element_type=jnp.float32)
    v = jax.lax.dot_general(trg, wv_ref[...], dn_proj,
                            preferred_element_type=jnp.float32)
    v2 = v * v
    q = (q * (1.0 / jnp.sqrt(jnp.float32(C)))).astype(jnp.bfloat16)
    kb = k.astype(jnp.bfloat16)
    vb = v.astype(jnp.bfloat16)
    v2b = v2.astype(jnp.bfloat16)

    dn_qkt = (((1,), (1,)), ((), ()))
    acc_mu = jnp.zeros((1, C), jnp.float32)
    acc_var = jnp.zeros((1, C), jnp.float32)
    for i in range(Ts // _QBLK):
        qb = q[i * _QBLK:(i + 1) * _QBLK]
        s = jax.lax.dot_general(qb, kb, dn_qkt,
                                preferred_element_type=jnp.float32)
        s = s - jnp.max(s, axis=1, keepdims=True)
        e = jnp.exp(s)
        rinv = 1.0 / jnp.sum(e, axis=1, keepdims=True)
        eb = e.astype(jnp.bfloat16)
        m = jnp.dot(eb, vb, preferred_element_type=jnp.float32) * rinv
        e2 = jnp.dot(eb, v2b, preferred_element_type=jnp.float32) * rinv
        acc_mu = acc_mu + jnp.sum(m, axis=0, keepdims=True)
        acc_var = acc_var + jnp.sum(jnp.maximum(e2 - m * m, 0.0), axis=0,
                                    keepdims=True)

    mu = acc_mu * (1.0 / Ts)
    std = jnp.sqrt(acc_var * (1.0 / Ts))
    out_ref[0] = std.reshape(C, 1) * src_n + mu.reshape(C, 1)


def kernel(src, trg, Wq, Wk, Wv):
    B, C, Ts = src.shape
    Tt = trg.shape[2]
    return pl.pallas_call(
        _can_kernel,
        out_shape=jax.ShapeDtypeStruct((B, C, Ts), src.dtype),
        grid=(B,),
        in_specs=[
            pl.BlockSpec((1, C, Ts), lambda b: (b, 0, 0)),
            pl.BlockSpec((1, C, Tt), lambda b: (b, 0, 0)),
            pl.BlockSpec((C, C), lambda b: (0, 0)),
            pl.BlockSpec((C, C), lambda b: (0, 0)),
            pl.BlockSpec((C, C), lambda b: (0, 0)),
        ],
        out_specs=pl.BlockSpec((1, C, Ts), lambda b: (b, 0, 0)),
        compiler_params=pltpu.CompilerParams(
            dimension_semantics=("parallel",),
            vmem_limit_bytes=64 * 1024 * 1024,
        ),
        name="chan_adaptive_norm",
    )(src, trg, Wq, Wk, Wv)


# speedup vs baseline: 5.9797x; 1.1554x over previous
import jax
import jax.numpy as jnp
from jax.experimental import pallas as pl
from jax.experimental.pallas import tpu as pltpu

_EPS = 1e-5
_QBLK = 512


def _can_kernel(src_ref, trg_ref, wq_ref, wk_ref, wv_ref, out_ref):
    src = src_ref[0]
    trg = trg_ref[0]
    C, Ts = src.shape
    Tt = trg.shape[1]

    def inorm(x, n):
        mu = jnp.mean(x, axis=1, keepdims=True)
        d = x - mu
        var = jnp.sum(d * d, axis=1, keepdims=True) * (1.0 / (n - 1))
        return d / (jnp.sqrt(var) + _EPS)

    src_n = inorm(src, Ts)
    trg_n = inorm(trg, Tt)

    wqb = (wq_ref[...] * (1.0 / jnp.sqrt(jnp.float32(C)))).astype(jnp.bfloat16)
    wkb = wk_ref[...].astype(jnp.bfloat16)
    wvb = wv_ref[...].astype(jnp.bfloat16)

    dn_proj = (((0,), (1,)), ((), ()))
    q = jax.lax.dot_general(src_n.astype(jnp.bfloat16), wqb, dn_proj,
                            preferred_element_type=jnp.float32
                            ).astype(jnp.bfloat16)
    k = jax.lax.dot_general(trg_n.astype(jnp.bfloat16), wkb, dn_proj,
                            preferred_element_type=jnp.float32
                            ).astype(jnp.bfloat16)
    v = jax.lax.dot_general(trg.astype(jnp.bfloat16), wvb, dn_proj,
                            preferred_element_type=jnp.float32)
    vcat = jnp.concatenate([v.astype(jnp.bfloat16),
                            (v * v).astype(jnp.bfloat16)], axis=1)

    dn_qkt = (((1,), (1,)), ((), ()))
    acc_mu = jnp.zeros((1, C), jnp.float32)
    acc_var = jnp.zeros((1, C), jnp.float32)
    for i in range(Ts // _QBLK):
        qb = q[i * _QBLK:(i + 1) * _QBLK]
        s = jax.lax.dot_general(qb, k, dn_qkt,
                                preferred_element_type=jnp.float32)
        e = jnp.exp(s)
        rinv = 1.0 / jnp.sum(e, axis=1, keepdims=True)
        me = jnp.dot(e.astype(jnp.bfloat16), vcat,
                     preferred_element_type=jnp.float32)
        m = me[:, :C] * rinv
        e2 = me[:, C:] * rinv
        acc_mu = acc_mu + jnp.sum(m, axis=0, keepdims=True)
        acc_var = acc_var + jnp.sum(jnp.maximum(e2 - m * m, 0.0), axis=0,
                                    keepdims=True)

    mu = acc_mu * (1.0 / Ts)
    std = jnp.sqrt(acc_var * (1.0 / Ts))
    out_ref[0] = std.reshape(C, 1) * src_n + mu.reshape(C, 1)


def kernel(src, trg, Wq, Wk, Wv):
    B, C, Ts = src.shape
    Tt = trg.shape[2]
    return pl.pallas_call(
        _can_kernel,
        out_shape=jax.ShapeDtypeStruct((B, C, Ts), src.dtype),
        grid=(B,),
        in_specs=[
            pl.BlockSpec((1, C, Ts), lambda b: (b, 0, 0)),
            pl.BlockSpec((1, C, Tt), lambda b: (b, 0, 0)),
            pl.BlockSpec((C, C), lambda b: (0, 0)),
            pl.BlockSpec((C, C), lambda b: (0, 0)),
            pl.BlockSpec((C, C), lambda b: (0, 0)),
        ],
        out_specs=pl.BlockSpec((1, C, Ts), lambda b: (b, 0, 0)),
        compiler_params=pltpu.CompilerParams(
            dimension_semantics=("parallel",),
            vmem_limit_bytes=64 * 1024 * 1024,
        ),
        name="chan_adaptive_norm",
    )(src, trg, Wq, Wk, Wv)
